# baseline (speedup 1.0000x reference)
import functools

import jax
import jax.numpy as jnp
from jax import lax
from jax.experimental import pallas as pl
from jax.experimental.pallas import tpu as pltpu, tpu_sc as plsc

N = 10000
D = 128
G = 64
E = 320000

NC = 2
NS = 16
NW = NC * NS

P = 10240
PAD_ROWS = P - N
C = 32
NCH = 320
NPH = 8
CPP = NCH // NPH
NBUF = 8
LAG = NBUF // 2
EPAD = NW * NCH * C
ROWS_PER_TILE = P // NS
ZROWS = C
ZCOPIES = ROWS_PER_TILE // ZROWS

DEG_RING = 8

_mesh = plsc.VectorSubcoreMesh(
    core_axis_name="c", subcore_axis_name="s", num_cores=NC, num_subcores=NS
)


@functools.partial(
    pl.kernel,
    out_type=jax.ShapeDtypeStruct((NC, P), jnp.float32),
    mesh=_mesh,
    scratch_types=[
        pltpu.VMEM((NCH, C), jnp.int32),
        pltpu.VMEM((C,), jnp.float32),
        pltpu.VMEM((ROWS_PER_TILE,), jnp.float32),
        pltpu.VMEM_SHARED((P,), jnp.float32),
        pltpu.SemaphoreType.DMA((DEG_RING,)),
    ],
)
def _deg_sc(dst_hbm, out_hbm, dst_v, ones_v, zero_v, deg_s, sems):
    c = lax.axis_index("c")
    s = lax.axis_index("s")
    wid = c * NS + s

    @pl.loop(0, ROWS_PER_TILE // 16)
    def _(i):
        zero_v[pl.ds(i * 16, 16)] = jnp.zeros((16,), jnp.float32)

    @pl.loop(0, C // 16)
    def _(i):
        ones_v[pl.ds(i * 16, 16)] = jnp.ones((16,), jnp.float32)

    pltpu.sync_copy(zero_v, deg_s.at[pl.ds(s * ROWS_PER_TILE, ROWS_PER_TILE)])
    plsc.subcore_barrier()

    pltpu.sync_copy(dst_hbm.at[wid], dst_v)

    @pl.loop(0, NCH, step=DEG_RING)
    def _(j):
        for k in range(DEG_RING):
            jj = j + k

            @pl.when(jj >= DEG_RING)
            def _():
                pltpu.make_async_copy(
                    ones_v, deg_s.at[dst_v.at[jj - DEG_RING]], sems.at[k]
                ).wait()

            pltpu.async_copy(ones_v, deg_s.at[dst_v.at[jj]], sems.at[k],
                             add=True)

    for k in range(DEG_RING):
        jj = NCH - DEG_RING + k
        pltpu.make_async_copy(
            ones_v, deg_s.at[dst_v.at[jj]], sems.at[k]).wait()

    plsc.subcore_barrier()
    pltpu.sync_copy(
        deg_s.at[pl.ds(s * ROWS_PER_TILE, ROWS_PER_TILE)],
        out_hbm.at[c, pl.ds(s * ROWS_PER_TILE, ROWS_PER_TILE)],
    )


@functools.partial(
    pl.kernel,
    out_type=jax.ShapeDtypeStruct((NC, P, D), jnp.float32),
    mesh=_mesh,
    scratch_types=[
        pltpu.VMEM((CPP, C), jnp.int32),
        pltpu.VMEM((CPP, C), jnp.int32),
        pltpu.VMEM((NBUF, C, D), jnp.float32),
        pltpu.VMEM_SHARED((P, D), jnp.float32),
        pltpu.SemaphoreType.DMA((NBUF,)),
        pltpu.SemaphoreType.DMA((NBUF,)),
    ],
)
def _agg_sc(h_hbm, src_hbm, dst_hbm, out_hbm, src_v, dst_v, rows_v,
            acc_s, sem_g, sem_s):
    c = lax.axis_index("c")
    s = lax.axis_index("s")
    wid = c * NS + s

    @pl.loop(0, C * D // 16)
    def _(i):
        rows_v[0, i // (D // 16), pl.ds((i % (D // 16)) * 16, 16)] = (
            jnp.zeros((16,), jnp.float32))

    for t in range(ZCOPIES):
        pltpu.sync_copy(
            rows_v.at[0], acc_s.at[pl.ds((s * ZCOPIES + t) * ZROWS, ZROWS)])
    plsc.subcore_barrier()

    for ph in range(NPH):
        pltpu.sync_copy(src_hbm.at[wid, pl.ds(ph * CPP, CPP)], src_v)
        pltpu.sync_copy(dst_hbm.at[wid, pl.ds(ph * CPP, CPP)], dst_v)

        for k in range(NBUF):
            pltpu.async_copy(h_hbm.at[src_v.at[k]], rows_v.at[k], sem_g.at[k])

        @pl.loop(0, CPP, step=NBUF)
        def _(j):
            for k in range(NBUF):
                jj = j + k
                pltpu.make_async_copy(
                    h_hbm.at[src_v.at[jj]], rows_v.at[k], sem_g.at[k]).wait()

                @pl.when(jj + NBUF < CPP)
                def _():
                    pltpu.async_copy(
                        h_hbm.at[src_v.at[jj + NBUF]], rows_v.at[k],
                        sem_g.at[k])

    plsc.subcore_barrier()
    pltpu.sync_copy(
        acc_s.at[pl.ds(s * ROWS_PER_TILE, ROWS_PER_TILE)],
        out_hbm.at[c, pl.ds(s * ROWS_PER_TILE, ROWS_PER_TILE)],
    )


BLK = 1024
NBLK = P // BLK


def _tc_prep_body(x_ref, w_ref, degt_ref, ht_ref, dinv_ref):
    dsum = degt_ref[:, 0:1] + degt_ref[:, 1:2] + 1.0
    dinv = lax.rsqrt(dsum)
    p = jnp.dot(x_ref[...], w_ref[...], preferred_element_type=jnp.float32)
    ht_ref[...] = p * dinv
    dinv_ref[...] = dinv


def _tc_layer_body(ap_ref, hprev_ref, dinv_ref, b_ref, w_ref, hnext_ref):
    acc = ap_ref[0] + ap_ref[1] + hprev_ref[...]
    act = jnp.maximum(acc * dinv_ref[...] + b_ref[...], 0.0)
    hnext_ref[...] = jnp.dot(
        act, w_ref[...], preferred_element_type=jnp.float32) * dinv_ref[...]


def _tc_head_body(ap_ref, hprev_ref, dinv_ref, b_ref, wl_ref, bl_ref,
                  out_ref, gmax_ref):
    i = pl.program_id(0)
    acc = ap_ref[0] + ap_ref[1] + hprev_ref[...]
    act = jnp.maximum(acc * dinv_ref[...] + b_ref[...], 0.0)
    rows = lax.broadcasted_iota(jnp.int32, (BLK, 1), 0) + i * BLK
    act = jnp.where(rows < N, act, 0.0)
    m = jnp.max(act, axis=0, keepdims=True)

    @pl.when(i == 0)
    def _():
        gmax_ref[...] = m

    @pl.when(i > 0)
    def _():
        gmax_ref[...] = jnp.maximum(gmax_ref[...], m)

    @pl.when(i == NBLK - 1)
    def _():
        out_ref[...] = jnp.dot(
            gmax_ref[...], wl_ref[...],
            preferred_element_type=jnp.float32) + bl_ref[...]


_tc_prep = pl.pallas_call(
    _tc_prep_body,
    grid=(NBLK,),
    in_specs=[
        pl.BlockSpec((BLK, D), lambda i: (i, 0)),
        pl.BlockSpec((D, D), lambda i: (0, 0)),
        pl.BlockSpec((BLK, NC), lambda i: (i, 0)),
    ],
    out_specs=[
        pl.BlockSpec((BLK, D), lambda i: (i, 0)),
        pl.BlockSpec((BLK, 1), lambda i: (i, 0)),
    ],
    out_shape=[
        jax.ShapeDtypeStruct((P, D), jnp.float32),
        jax.ShapeDtypeStruct((P, 1), jnp.float32),
    ],
)

_tc_layer = pl.pallas_call(
    _tc_layer_body,
    grid=(NBLK,),
    in_specs=[
        pl.BlockSpec((NC, BLK, D), lambda i: (0, i, 0)),
        pl.BlockSpec((BLK, D), lambda i: (i, 0)),
        pl.BlockSpec((BLK, 1), lambda i: (i, 0)),
        pl.BlockSpec((1, D), lambda i: (0, 0)),
        pl.BlockSpec((D, D), lambda i: (0, 0)),
    ],
    out_specs=pl.BlockSpec((BLK, D), lambda i: (i, 0)),
    out_shape=jax.ShapeDtypeStruct((P, D), jnp.float32),
)

_tc_head = pl.pallas_call(
    _tc_head_body,
    grid=(NBLK,),
    in_specs=[
        pl.BlockSpec((NC, BLK, D), lambda i: (0, i, 0)),
        pl.BlockSpec((BLK, D), lambda i: (i, 0)),
        pl.BlockSpec((BLK, 1), lambda i: (i, 0)),
        pl.BlockSpec((1, D), lambda i: (0, 0)),
        pl.BlockSpec((D, G), lambda i: (0, 0)),
        pl.BlockSpec((1, G), lambda i: (0, 0)),
    ],
    out_specs=pl.BlockSpec((1, G), lambda i: (0, 0)),
    out_shape=jax.ShapeDtypeStruct((1, G), jnp.float32),
    scratch_shapes=[pltpu.VMEM((1, D), jnp.float32)],
)


def kernel(x, edge_index, W1, b1, W2, b2, W3, b3, Wl, bl):
    src = edge_index[0].astype(jnp.int32)
    dst = edge_index[1].astype(jnp.int32)
    pad = N + (jnp.arange(EPAD - E, dtype=jnp.int32) % PAD_ROWS)
    src3 = jnp.concatenate([src, pad]).reshape(NW, NCH, C)
    dst3 = jnp.concatenate([dst, pad]).reshape(NW, NCH, C)
    x_pad = jnp.pad(x, ((0, P - N), (0, 0)))

    degp = _deg_sc(dst3)
    ht1, dinv = _tc_prep(x_pad, W1, degp.T)
    a1 = _agg_sc(ht1, src3, dst3)
    ht2 = _tc_layer(a1, ht1, dinv, b1.reshape(1, D), W2)
    a2 = _agg_sc(ht2, src3, dst3)
    ht3 = _tc_layer(a2, ht2, dinv, b2.reshape(1, D), W3)
    a3 = _agg_sc(ht3, src3, dst3)
    out = _tc_head(a3, ht3, dinv, b3.reshape(1, D), Wl, bl.reshape(1, G))
    return out.reshape(G)


# speedup vs baseline: 1.0117x; 1.0117x over previous
import functools

import jax
import jax.numpy as jnp
from jax import lax
from jax.experimental import pallas as pl
from jax.experimental.pallas import tpu as pltpu, tpu_sc as plsc

N = 10000
D = 128
G = 64
E = 320000

NC = 2
NS = 16
NW = NC * NS

TRASH = 240
PACC = N + TRASH
C = 64
NCH = 160
EPAD = NW * NCH * C
NBUF = 5
GL = 3
SL = 2
NIDX = 10
IL = 6
STEP = 10

ZTILE = PACC // NS
DEG_RING = 8

_mesh = plsc.VectorSubcoreMesh(
    core_axis_name="c", subcore_axis_name="s", num_cores=NC, num_subcores=NS
)


@functools.partial(
    pl.kernel,
    out_type=jax.ShapeDtypeStruct((NC, 10240), jnp.float32),
    mesh=_mesh,
    scratch_types=[
        pltpu.VMEM((NCH, C), jnp.int32),
        pltpu.VMEM((C,), jnp.float32),
        pltpu.VMEM((640,), jnp.float32),
        pltpu.VMEM_SHARED((10240,), jnp.float32),
        pltpu.SemaphoreType.DMA((DEG_RING,)),
    ],
)
def _deg_sc(dst_hbm, out_hbm, dst_v, ones_v, zero_v, deg_s, sems):
    c = lax.axis_index("c")
    s = lax.axis_index("s")
    wid = c * NS + s

    @pl.loop(0, 640 // 16)
    def _(i):
        zero_v[pl.ds(i * 16, 16)] = jnp.zeros((16,), jnp.float32)

    @pl.loop(0, C // 16)
    def _(i):
        ones_v[pl.ds(i * 16, 16)] = jnp.ones((16,), jnp.float32)

    pltpu.sync_copy(zero_v, deg_s.at[pl.ds(s * 640, 640)])
    plsc.subcore_barrier()

    pltpu.sync_copy(dst_hbm.at[wid], dst_v)

    @pl.loop(0, NCH, step=DEG_RING)
    def _(j):
        for k in range(DEG_RING):
            jj = j + k

            @pl.when(jj >= DEG_RING)
            def _():
                pltpu.make_async_copy(
                    ones_v, deg_s.at[dst_v.at[jj - DEG_RING]], sems.at[k]
                ).wait()

            pltpu.async_copy(ones_v, deg_s.at[dst_v.at[jj]], sems.at[k],
                             add=True)

    for k in range(DEG_RING):
        jj = NCH - DEG_RING + k
        pltpu.make_async_copy(
            ones_v, deg_s.at[dst_v.at[jj]], sems.at[k]).wait()

    plsc.subcore_barrier()
    pltpu.sync_copy(
        deg_s.at[pl.ds(s * 640, 640)],
        out_hbm.at[c, pl.ds(s * 640, 640)],
    )


@functools.partial(
    pl.kernel,
    out_type=jax.ShapeDtypeStruct((NC, PACC, D), jnp.float32),
    mesh=_mesh,
    scratch_types=[
        pltpu.VMEM((NIDX, 2, C), jnp.int32),
        pltpu.VMEM((NBUF, C, D), jnp.float32),
        pltpu.VMEM_SHARED((PACC, D), jnp.float32),
        pltpu.SemaphoreType.DMA((NIDX,)),
        pltpu.SemaphoreType.DMA((NBUF,)),
        pltpu.SemaphoreType.DMA((NBUF,)),
    ],
)
def _agg_sc(h_hbm, e_hbm, out_hbm, idx_v, rows_v, acc_s, sem_i, sem_g, sem_s):
    c = lax.axis_index("c")
    s = lax.axis_index("s")
    wid = c * NS + s

    @pl.loop(0, C * D // 16)
    def _(i):
        rows_v[0, i // (D // 16), pl.ds((i % (D // 16)) * 16, 16)] = (
            jnp.zeros((16,), jnp.float32))

    for t in range(ZTILE // C):
        pltpu.sync_copy(
            rows_v.at[0], acc_s.at[pl.ds(s * ZTILE + t * C, C)])
    plsc.subcore_barrier()

    for t in range(IL):
        pltpu.async_copy(e_hbm.at[wid, t], idx_v.at[t], sem_i.at[t])
    for t in range(GL):
        pltpu.make_async_copy(
            e_hbm.at[wid, t], idx_v.at[t], sem_i.at[t]).wait()
        pltpu.async_copy(
            h_hbm.at[idx_v.at[t, 0]], rows_v.at[t], sem_g.at[t])

    @pl.loop(0, NCH, step=STEP)
    def _(j):
        for k in range(STEP):
            jj = j + k
            ri = k % NBUF
            rg = (k + GL) % NBUF
            rs = (k - SL) % NBUF
            qi = k % NIDX
            qg = (k + GL) % NIDX
            qs = (k - SL) % NIDX
            qp = (k + IL) % NIDX

            @pl.when(jj >= SL)
            def _():
                pltpu.make_async_copy(
                    rows_v.at[rs], acc_s.at[idx_v.at[qs, 1]],
                    sem_s.at[rs]).wait()

            @pl.when(jj + GL < NCH)
            def _():
                pltpu.make_async_copy(
                    e_hbm.at[wid, jj + GL], idx_v.at[qg], sem_i.at[qg]).wait()
                pltpu.async_copy(
                    h_hbm.at[idx_v.at[qg, 0]], rows_v.at[rg], sem_g.at[rg])

            pltpu.make_async_copy(
                h_hbm.at[idx_v.at[qi, 0]], rows_v.at[ri], sem_g.at[ri]).wait()
            pltpu.async_copy(
                rows_v.at[ri], acc_s.at[idx_v.at[qi, 1]], sem_s.at[ri],
                add=True)

            @pl.when(jj + IL < NCH)
            def _():
                pltpu.async_copy(
                    e_hbm.at[wid, jj + IL], idx_v.at[qp], sem_i.at[qp])

    for i in range(SL):
        jj = NCH - SL + i
        pltpu.make_async_copy(
            rows_v.at[jj % NBUF], acc_s.at[idx_v.at[jj % NIDX, 1]],
            sem_s.at[jj % NBUF]).wait()

    plsc.subcore_barrier()
    pltpu.sync_copy(
        acc_s.at[pl.ds(s * ZTILE, ZTILE)],
        out_hbm.at[c, pl.ds(s * ZTILE, ZTILE)],
    )


BLK = 1000
NBLK = N // BLK


def _tc_prep_body(x_ref, w_ref, degt_ref, ht_ref, dinv_ref):
    dsum = degt_ref[:, 0:1] + degt_ref[:, 1:2] + 1.0
    dinv = lax.rsqrt(dsum)
    p = jnp.dot(x_ref[...], w_ref[...], preferred_element_type=jnp.float32)
    ht_ref[...] = p * dinv
    dinv_ref[...] = dinv


def _tc_layer_body(ap_ref, hprev_ref, dinv_ref, b_ref, w_ref, hnext_ref):
    acc = ap_ref[0] + ap_ref[1] + hprev_ref[...]
    act = jnp.maximum(acc * dinv_ref[...] + b_ref[...], 0.0)
    hnext_ref[...] = jnp.dot(
        act, w_ref[...], preferred_element_type=jnp.float32) * dinv_ref[...]


def _tc_head_body(ap_ref, hprev_ref, dinv_ref, b_ref, wl_ref, bl_ref,
                  out_ref, gmax_ref):
    i = pl.program_id(0)
    acc = ap_ref[0] + ap_ref[1] + hprev_ref[...]
    act = jnp.maximum(acc * dinv_ref[...] + b_ref[...], 0.0)
    m = jnp.max(act, axis=0, keepdims=True)

    @pl.when(i == 0)
    def _():
        gmax_ref[...] = m

    @pl.when(i > 0)
    def _():
        gmax_ref[...] = jnp.maximum(gmax_ref[...], m)

    @pl.when(i == NBLK - 1)
    def _():
        out_ref[...] = jnp.dot(
            gmax_ref[...], wl_ref[...],
            preferred_element_type=jnp.float32) + bl_ref[...]


_tc_prep = pl.pallas_call(
    _tc_prep_body,
    grid=(NBLK,),
    in_specs=[
        pl.BlockSpec((BLK, D), lambda i: (i, 0)),
        pl.BlockSpec((D, D), lambda i: (0, 0)),
        pl.BlockSpec((BLK, NC), lambda i: (i, 0)),
    ],
    out_specs=[
        pl.BlockSpec((BLK, D), lambda i: (i, 0)),
        pl.BlockSpec((BLK, 1), lambda i: (i, 0)),
    ],
    out_shape=[
        jax.ShapeDtypeStruct((N, D), jnp.float32),
        jax.ShapeDtypeStruct((N, 1), jnp.float32),
    ],
)

_tc_layer = pl.pallas_call(
    _tc_layer_body,
    grid=(NBLK,),
    in_specs=[
        pl.BlockSpec((NC, BLK, D), lambda i: (0, i, 0)),
        pl.BlockSpec((BLK, D), lambda i: (i, 0)),
        pl.BlockSpec((BLK, 1), lambda i: (i, 0)),
        pl.BlockSpec((1, D), lambda i: (0, 0)),
        pl.BlockSpec((D, D), lambda i: (0, 0)),
    ],
    out_specs=pl.BlockSpec((BLK, D), lambda i: (i, 0)),
    out_shape=jax.ShapeDtypeStruct((N, D), jnp.float32),
)

_tc_head = pl.pallas_call(
    _tc_head_body,
    grid=(NBLK,),
    in_specs=[
        pl.BlockSpec((NC, BLK, D), lambda i: (0, i, 0)),
        pl.BlockSpec((BLK, D), lambda i: (i, 0)),
        pl.BlockSpec((BLK, 1), lambda i: (i, 0)),
        pl.BlockSpec((1, D), lambda i: (0, 0)),
        pl.BlockSpec((D, G), lambda i: (0, 0)),
        pl.BlockSpec((1, G), lambda i: (0, 0)),
    ],
    out_specs=pl.BlockSpec((1, G), lambda i: (0, 0)),
    out_shape=jax.ShapeDtypeStruct((1, G), jnp.float32),
    scratch_shapes=[pltpu.VMEM((1, D), jnp.float32)],
)


def kernel(x, edge_index, W1, b1, W2, b2, W3, b3, Wl, bl):
    src = edge_index[0].astype(jnp.int32)
    dst = edge_index[1].astype(jnp.int32)
    npad = EPAD - E
    pad_src = jnp.arange(npad, dtype=jnp.int32) % N
    pad_dst = N + (jnp.arange(npad, dtype=jnp.int32) % TRASH)
    srcp = jnp.concatenate([src, pad_src]).reshape(NW, NCH, 1, C)
    dstp = jnp.concatenate([dst, pad_dst]).reshape(NW, NCH, 1, C)
    eint = jnp.concatenate([srcp, dstp], axis=2)
    dst3 = dstp.reshape(NW, NCH, C)

    degp = _deg_sc(dst3)
    ht1, dinv = _tc_prep(x, W1, degp.T[:N])
    a1 = _agg_sc(ht1, eint)
    ht2 = _tc_layer(a1, ht1, dinv, b1.reshape(1, D), W2)
    a2 = _agg_sc(ht2, eint)
    ht3 = _tc_layer(a2, ht2, dinv, b2.reshape(1, D), W3)
    a3 = _agg_sc(ht3, eint)
    out = _tc_head(a3, ht3, dinv, b3.reshape(1, D), Wl, bl.reshape(1, G))
    return out.reshape(G)
